# baseline (speedup 1.0000x reference)
import jax
import jax.numpy as jnp
from jax.experimental import pallas as pl
from jax.experimental.pallas import tpu as pltpu

_STATE = 128
_N_IO = 320000
_N_VAL = 160000
_IO_BLK = 2000
_VAL_BLK = 1000
_STEPS = _N_IO // _IO_BLK


def _pool_kernel(io_ref, val_ref, w_ref, b_ref, out_ref, io_acc, val_acc):
    i = pl.program_id(0)

    @pl.when(i == 0)
    def _init():
        io_acc[...] = jnp.zeros_like(io_acc)
        val_acc[...] = jnp.zeros_like(val_acc)

    io_acc[...] += jnp.sum(io_ref[...], axis=0, keepdims=True)
    val_acc[...] += jnp.sum(val_ref[...], axis=0, keepdims=True)

    @pl.when(i == _STEPS - 1)
    def _finish():
        io_mean = io_acc[...] / _N_IO
        val_mean = val_acc[...] / _N_VAL
        joint = jnp.concatenate([io_mean, val_mean], axis=1)
        out_ref[...] = (
            jax.lax.dot_general(
                joint, w_ref[...],
                (((1,), (1,)), ((), ())),
                preferred_element_type=jnp.float32,
            )
            + b_ref[...]
        )


def kernel(io_embed, value_embed, W, b):
    b2 = b.reshape(1, _STATE)
    out = pl.pallas_call(
        _pool_kernel,
        grid=(_STEPS,),
        in_specs=[
            pl.BlockSpec((_IO_BLK, 2 * _STATE), lambda i: (i, 0)),
            pl.BlockSpec((_VAL_BLK, _STATE), lambda i: (i, 0)),
            pl.BlockSpec((_STATE, 3 * _STATE), lambda i: (0, 0)),
            pl.BlockSpec((1, _STATE), lambda i: (0, 0)),
        ],
        out_specs=pl.BlockSpec((1, _STATE), lambda i: (0, 0)),
        out_shape=jax.ShapeDtypeStruct((1, _STATE), jnp.float32),
        scratch_shapes=[
            pltpu.VMEM((1, 2 * _STATE), jnp.float32),
            pltpu.VMEM((1, _STATE), jnp.float32),
        ],
    )(io_embed, value_embed, W, b2)
    return out


# speedup vs baseline: 1.4301x; 1.4301x over previous
import jax
import jax.numpy as jnp
from jax.experimental import pallas as pl
from jax.experimental.pallas import tpu as pltpu

_STATE = 128
_N_IO = 320000
_N_VAL = 160000
_IO_BLK = 8000
_VAL_BLK = 4000
_STEPS = _N_IO // _IO_BLK


def _pool_kernel(io_ref, val_ref, w_ref, b_ref, out_ref, io_acc, val_acc):
    i = pl.program_id(0)

    @pl.when(i == 0)
    def _init():
        io_acc[...] = jnp.zeros_like(io_acc)
        val_acc[...] = jnp.zeros_like(val_acc)

    io_acc[...] += jnp.sum(io_ref[...], axis=0, keepdims=True)
    val_acc[...] += jnp.sum(val_ref[...], axis=0, keepdims=True)

    @pl.when(i == _STEPS - 1)
    def _finish():
        io_mean = io_acc[...] / _N_IO
        val_mean = val_acc[...] / _N_VAL
        joint = jnp.concatenate([io_mean, val_mean], axis=1)
        out_ref[...] = (
            jax.lax.dot_general(
                joint, w_ref[...],
                (((1,), (1,)), ((), ())),
                preferred_element_type=jnp.float32,
            )
            + b_ref[...]
        )


def kernel(io_embed, value_embed, W, b):
    b2 = b.reshape(1, _STATE)
    out = pl.pallas_call(
        _pool_kernel,
        grid=(_STEPS,),
        in_specs=[
            pl.BlockSpec((_IO_BLK, 2 * _STATE), lambda i: (i, 0)),
            pl.BlockSpec((_VAL_BLK, _STATE), lambda i: (i, 0)),
            pl.BlockSpec((_STATE, 3 * _STATE), lambda i: (0, 0)),
            pl.BlockSpec((1, _STATE), lambda i: (0, 0)),
        ],
        out_specs=pl.BlockSpec((1, _STATE), lambda i: (0, 0)),
        out_shape=jax.ShapeDtypeStruct((1, _STATE), jnp.float32),
        scratch_shapes=[
            pltpu.VMEM((1, 2 * _STATE), jnp.float32),
            pltpu.VMEM((1, _STATE), jnp.float32),
        ],
    )(io_embed, value_embed, W, b2)
    return out
